# baseline (speedup 1.0000x reference)
import functools

import jax
import jax.numpy as jnp
from jax import lax
from jax.experimental import pallas as pl
from jax.experimental.pallas import tpu as pltpu
from jax.experimental.pallas import tpu_sc as plsc

NUM_TABLES = 26
VOCAB = 100000
EMBED_DIM = 16
BATCH = 1024
HIST = 20

TOTAL = NUM_TABLES * BATCH * HIST
CHUNK = 1280
CHUNKS_PER_TABLE = (BATCH * HIST) // CHUNK
NUM_CHUNKS = TOTAL // CHUNK


@functools.lru_cache(maxsize=None)
def _build():
    info = plsc.get_sparse_core_info()
    nc, ns = info.num_cores, info.num_subcores
    nw = nc * ns
    chunks_per_w = NUM_CHUNKS // nw

    mesh = plsc.VectorSubcoreMesh(core_axis_name="c", subcore_axis_name="s")

    @functools.partial(
        pl.kernel,
        mesh=mesh,
        out_type=jax.ShapeDtypeStruct((TOTAL, EMBED_DIM), jnp.float32),
        scratch_types=[
            pltpu.VMEM((CHUNK,), jnp.int32),
            pltpu.VMEM((CHUNK, EMBED_DIM), jnp.float32),
            pltpu.SemaphoreType.DMA,
        ],
        compiler_params=pltpu.CompilerParams(use_tc_tiling_on_sc=False),
    )
    def gather_kernel(idx_hbm, tables_hbm, out_hbm, idx_v, rows_v, sem):
        wid = lax.axis_index("s") * nc + lax.axis_index("c")
        for i in range(chunks_per_w):
            c = wid * chunks_per_w + i
            t = c // CHUNKS_PER_TABLE
            base = c * CHUNK
            pltpu.sync_copy(idx_hbm.at[pl.ds(base, CHUNK)], idx_v)
            pltpu.async_copy(tables_hbm.at[t].at[idx_v], rows_v, sem).wait()
            pltpu.sync_copy(rows_v, out_hbm.at[pl.ds(base, CHUNK)])

    return gather_kernel


def kernel(indices, tables):
    idx_flat = indices.reshape(-1).astype(jnp.int32)
    out = _build()(idx_flat, tables)
    return out.reshape(NUM_TABLES, BATCH, HIST, EMBED_DIM)


# speedup vs baseline: 1.0091x; 1.0091x over previous
import functools

import jax
import jax.numpy as jnp
from jax import lax
from jax.experimental import pallas as pl
from jax.experimental.pallas import tpu as pltpu
from jax.experimental.pallas import tpu_sc as plsc

NUM_TABLES = 26
VOCAB = 100000
EMBED_DIM = 16
BATCH = 1024
HIST = 20

TOTAL = NUM_TABLES * BATCH * HIST
CHUNK = 1280
CHUNKS_PER_TABLE = (BATCH * HIST) // CHUNK
NUM_CHUNKS = TOTAL // CHUNK


NBUF = 4


@functools.lru_cache(maxsize=None)
def _build():
    info = plsc.get_sparse_core_info()
    nc, ns = info.num_cores, info.num_subcores
    nw = nc * ns
    chunks_per_w = NUM_CHUNKS // nw
    ids_per_w = chunks_per_w * CHUNK

    mesh = plsc.VectorSubcoreMesh(core_axis_name="c", subcore_axis_name="s")

    @functools.partial(
        pl.kernel,
        mesh=mesh,
        out_type=jax.ShapeDtypeStruct((TOTAL, EMBED_DIM), jnp.float32),
        scratch_types=[
            pltpu.VMEM((ids_per_w,), jnp.int32),
            [pltpu.VMEM((CHUNK, EMBED_DIM), jnp.float32) for _ in range(NBUF)],
            pltpu.SemaphoreType.DMA,
            pltpu.SemaphoreType.DMA,
        ],
        compiler_params=pltpu.CompilerParams(use_tc_tiling_on_sc=False),
    )
    def gather_kernel(idx_hbm, tables_hbm, out_hbm, idx_v, rows, sem_g, sem_o):
        wid = lax.axis_index("s") * nc + lax.axis_index("c")
        c0 = wid * chunks_per_w
        pltpu.sync_copy(idx_hbm.at[pl.ds(c0 * CHUNK, ids_per_w)], idx_v)

        cps_g = [None] * chunks_per_w
        cps_o = [None] * chunks_per_w
        for i in range(chunks_per_w):
            if i >= NBUF:
                cps_o[i - NBUF].wait()
            c = c0 + i
            t = c // CHUNKS_PER_TABLE
            cps_g[i] = pltpu.async_copy(
                tables_hbm.at[t].at[idx_v.at[pl.ds(i * CHUNK, CHUNK)]],
                rows[i % NBUF], sem_g)
            if i >= 1:
                cps_g[i - 1].wait()
                cps_o[i - 1] = pltpu.async_copy(
                    rows[(i - 1) % NBUF],
                    out_hbm.at[pl.ds((c - 1) * CHUNK, CHUNK)], sem_o)
        last = chunks_per_w - 1
        cps_g[last].wait()
        cps_o[last] = pltpu.async_copy(
            rows[last % NBUF], out_hbm.at[pl.ds((c0 + last) * CHUNK, CHUNK)],
            sem_o)
        for i in range(chunks_per_w - NBUF, chunks_per_w):
            cps_o[i].wait()

    return gather_kernel


def kernel(indices, tables):
    idx_flat = indices.reshape(-1).astype(jnp.int32)
    out = _build()(idx_flat, tables)
    return out.reshape(NUM_TABLES, BATCH, HIST, EMBED_DIM)


# speedup vs baseline: 6.0931x; 6.0379x over previous
import functools

import jax
import jax.numpy as jnp
from jax import lax
from jax.experimental import pallas as pl
from jax.experimental.pallas import tpu as pltpu
from jax.experimental.pallas import tpu_sc as plsc

NUM_TABLES = 26
VOCAB = 100000
EMBED_DIM = 16
BATCH = 1024
HIST = 20

TOTAL = NUM_TABLES * BATCH * HIST
NUM_PAIRS = NUM_TABLES * EMBED_DIM
IDS_PER_TABLE = BATCH * HIST
OUT_ROWS = NUM_TABLES * HIST * EMBED_DIM


@functools.lru_cache(maxsize=None)
def _build():
    info = plsc.get_sparse_core_info()
    nc, ns, nl = info.num_cores, info.num_subcores, info.num_lanes
    nw = nc * ns
    pairs_per_w = NUM_PAIRS // nw
    groups = BATCH // nl

    mesh = plsc.VectorSubcoreMesh(core_axis_name="c", subcore_axis_name="s")

    @functools.partial(
        pl.kernel,
        mesh=mesh,
        out_type=jax.ShapeDtypeStruct((OUT_ROWS, BATCH), jnp.float32),
        scratch_types=[
            pltpu.VMEM((VOCAB,), jnp.float32),
            pltpu.VMEM((IDS_PER_TABLE,), jnp.int32),
            [pltpu.VMEM((BATCH,), jnp.float32) for _ in range(2)],
            pltpu.SemaphoreType.DMA,
            pltpu.SemaphoreType.DMA,
            pltpu.SemaphoreType.DMA,
        ],
        compiler_params=pltpu.CompilerParams(needs_layout_passes=False),
    )
    def gather_kernel(idx_hbm, tbl_hbm, out_hbm, tbl_v, idx_v, obufs,
                      sem_t, sem_i, sem_o):
        wid = lax.axis_index("s") * nc + lax.axis_index("c")
        iota20 = lax.iota(jnp.int32, nl) * HIST

        for i in range(pairs_per_w):
            c = wid * pairs_per_w + i
            t = c // EMBED_DIM
            e = c % EMBED_DIM
            cp_t = pltpu.async_copy(tbl_hbm.at[t, e], tbl_v, sem_t)
            cp_i = pltpu.async_copy(
                idx_hbm.at[pl.ds(t * IDS_PER_TABLE, IDS_PER_TABLE)],
                idx_v, sem_i)
            cp_t.wait()
            cp_i.wait()
            orow0 = t * (HIST * EMBED_DIM) + e
            cps_o = [None, None]
            for h in range(HIST):
                ob = obufs[h % 2]
                if cps_o[h % 2] is not None:
                    cps_o[h % 2].wait()

                def body(g, _, h=h, ob=ob):
                    pvec = iota20 + (g * (nl * HIST) + h)
                    ids = plsc.load_gather(idx_v, [pvec])
                    vals = plsc.load_gather(tbl_v, [ids])
                    ob[pl.ds(g * nl, nl)] = vals
                    return 0

                lax.fori_loop(0, groups, body, 0)
                cps_o[h % 2] = pltpu.async_copy(
                    ob, out_hbm.at[orow0 + h * EMBED_DIM], sem_o)
            cps_o[0].wait()
            cps_o[1].wait()

    return gather_kernel


def kernel(indices, tables):
    idx_flat = indices.reshape(-1).astype(jnp.int32)
    tbl_t = tables.transpose(0, 2, 1)
    out = _build()(idx_flat, tbl_t)
    return out.reshape(NUM_TABLES, HIST, EMBED_DIM, BATCH).transpose(0, 3, 1, 2)


# speedup vs baseline: 6.8274x; 1.1205x over previous
import functools

import jax
import jax.numpy as jnp
from jax import lax
from jax.experimental import pallas as pl
from jax.experimental.pallas import tpu as pltpu
from jax.experimental.pallas import tpu_sc as plsc

NUM_TABLES = 26
VOCAB = 100000
EMBED_DIM = 16
BATCH = 1024
HIST = 20

IDS_PER_TABLE = BATCH * HIST
NUM_PAIRS = NUM_TABLES * EMBED_DIM
OUT_ROWS = NUM_TABLES * HIST * EMBED_DIM


@functools.lru_cache(maxsize=None)
def _build():
    info = plsc.get_sparse_core_info()
    nc, ns, nl = info.num_cores, info.num_subcores, info.num_lanes
    nw = nc * ns
    pairs_per_w = NUM_PAIRS // nw
    unroll = 4
    steps = BATCH // (nl * unroll)

    mesh = plsc.VectorSubcoreMesh(core_axis_name="c", subcore_axis_name="s")

    @functools.partial(
        pl.kernel,
        mesh=mesh,
        out_type=jax.ShapeDtypeStruct((OUT_ROWS, BATCH), jnp.float32),
        scratch_types=[
            pltpu.VMEM((VOCAB,), jnp.float32),
            pltpu.VMEM((IDS_PER_TABLE,), jnp.int32),
            [pltpu.VMEM((BATCH,), jnp.float32) for _ in range(2)],
            pltpu.SemaphoreType.DMA,
            pltpu.SemaphoreType.DMA,
            [pltpu.SemaphoreType.DMA for _ in range(2)],
        ],
        compiler_params=pltpu.CompilerParams(needs_layout_passes=False),
    )
    def gather_kernel(idx_hbm, tbl_hbm, out_hbm, tbl_v, idx_v, obufs,
                      sem_t, sem_i, sems_o):
        wid = lax.axis_index("s") * nc + lax.axis_index("c")

        def do_hist(h, orow0):
            ob = obufs[h % 2]
            base = h * BATCH

            def gbody(g, _):
                off = base + g * (nl * unroll)
                for k in range(unroll):
                    ids = idx_v[pl.ds(off + k * nl, nl)]
                    ob[pl.ds(g * (nl * unroll) + k * nl, nl)] = (
                        plsc.load_gather(tbl_v, [ids]))
                return 0

            lax.fori_loop(0, steps, gbody, 0)
            return pltpu.async_copy(ob, out_hbm.at[orow0 + h * EMBED_DIM],
                                    sems_o[h % 2])

        for i in range(pairs_per_w):
            c = wid * pairs_per_w + i
            t = c // EMBED_DIM
            e = c % EMBED_DIM
            cp_t = pltpu.async_copy(tbl_hbm.at[t, e], tbl_v, sem_t)
            cp_i = pltpu.async_copy(
                idx_hbm.at[pl.ds(t * IDS_PER_TABLE, IDS_PER_TABLE)],
                idx_v, sem_i)
            cp_t.wait()
            cp_i.wait()
            orow0 = t * (HIST * EMBED_DIM) + e
            cps_o = [do_hist(0, orow0), do_hist(1, orow0)]
            for h in range(2, HIST):
                cps_o[h % 2].wait()
                cps_o[h % 2] = do_hist(h, orow0)
            cps_o[0].wait()
            cps_o[1].wait()

    return gather_kernel


def kernel(indices, tables):
    idx_t = indices.transpose(0, 2, 1).reshape(-1).astype(jnp.int32)
    tbl_t = tables.transpose(0, 2, 1)
    out = _build()(idx_t, tbl_t)
    return out.reshape(NUM_TABLES, HIST, EMBED_DIM, BATCH).transpose(0, 3, 1, 2)
